# baseline (speedup 1.0000x reference)
import jax
import jax.numpy as jnp
from jax import lax
from jax.experimental import pallas as pl
from jax.experimental.pallas import tpu as pltpu
from jax.experimental.pallas import tpu_sc as plsc

N_ROWS = 16384
N_FIELDS = 26
FIELD_N = 100
ROW_W = N_FIELDS * FIELD_N
NC, NS = 2, 16
NW = NC * NS
ROWS_PER_W = N_ROWS // NW
BC = 200
BR = 128
SGS = ROW_W // BC
RTS = ROWS_PER_W // BR
CHUNKS = SGS * RTS
XW = ROWS_PER_W * N_FIELDS


NBUF = 4


def _body(x_hbm, out_hbm, xbuf, buf, sem0, sem1, sem2, sem3):
    wid = lax.axis_index("c") * NS + lax.axis_index("s")
    lanes = lax.iota(jnp.int32, 16)
    ones = jnp.full((16,), 1, jnp.int32)
    zeros = jnp.zeros((16,), jnp.int32)
    sems = (sem0, sem1, sem2, sem3)
    row0 = wid * ROWS_PER_W

    pltpu.sync_copy(x_hbm.at[:, pl.ds(row0, ROWS_PER_W)], xbuf)

    for b in range(NBUF):
        def zero_body(k, carry, b=b):
            buf[b, k >> 3, pl.ds((k & 7) * 16, 16)] = zeros
            return carry

        lax.fori_loop(0, BC * BR // 16, zero_body, 0)

    def scatter(c, b, val):
        sg = c >> 2
        rt = c & 3
        bsplat = jnp.full((16,), b, jnp.int32)
        for ii in range(2):
            fvec = jnp.full((16,), 2 * sg + ii, jnp.int32)
            for g in range(8):
                rloc = rt * BR + g * 16 + lanes
                xv = plsc.load_gather(xbuf, [fvec, rloc])
                plsc.store_scatter(
                    buf, [bsplat, ii * FIELD_N + xv, g * 16 + lanes], val)

    def start_stream(c, b):
        sg = c >> 2
        rt = c & 3
        pltpu.async_copy(
            buf.at[b],
            out_hbm.at[pl.ds(sg * BC, BC), pl.ds(row0 + rt * BR, BR)],
            sems[b])

    def wait_stream(b):
        pltpu.make_async_copy(
            buf.at[b],
            out_hbm.at[pl.ds(0, BC), pl.ds(0, BR)],
            sems[b]).wait()

    for b in range(NBUF):
        scatter(jnp.int32(b), b, ones)
        start_stream(jnp.int32(b), b)

    def loop_body(it, carry):
        for b in range(NBUF):
            c = it * NBUF + b
            wait_stream(b)
            scatter(c - NBUF, b, zeros)
            scatter(c, b, ones)
            start_stream(c, b)
        return carry

    lax.fori_loop(1, CHUNKS // NBUF, loop_body, 0)

    for b in range(NBUF):
        wait_stream(b)


@jax.jit
def _run(xt):
    mesh = plsc.VectorSubcoreMesh(core_axis_name="c", subcore_axis_name="s")
    f = pl.kernel(
        _body,
        out_type=jax.ShapeDtypeStruct((ROW_W, N_ROWS), jnp.int32),
        mesh=mesh,
        scratch_types=[
            pltpu.VMEM((N_FIELDS, ROWS_PER_W), jnp.int32),
            pltpu.VMEM((NBUF, BC, BR), jnp.int32),
            pltpu.SemaphoreType.DMA,
            pltpu.SemaphoreType.DMA,
            pltpu.SemaphoreType.DMA,
            pltpu.SemaphoreType.DMA,
        ],
        compiler_params=pltpu.CompilerParams(needs_layout_passes=False),
    )
    return f(xt)


def kernel(x):
    return jnp.transpose(_run(jnp.transpose(x)))


# speedup vs baseline: 1.1767x; 1.1767x over previous
import jax
import jax.numpy as jnp
from jax import lax
from jax.experimental import pallas as pl
from jax.experimental.pallas import tpu as pltpu
from jax.experimental.pallas import tpu_sc as plsc

N_ROWS = 16384
N_FIELDS = 26
FIELD_N = 100
ROW_W = N_FIELDS * FIELD_N
NC, NS = 2, 16
NW = NC * NS
ROWS_PER_W = N_ROWS // NW
BC = 200
BR = 128
SGS = ROW_W // BC
RTS = ROWS_PER_W // BR
CHUNKS = SGS * RTS
XW = ROWS_PER_W * N_FIELDS


NBUF = 2


def _body(x_hbm, out_hbm, xbuf, buf, sem0, sem1, sem2, sem3):
    wid = lax.axis_index("c") * NS + lax.axis_index("s")
    lanes = lax.iota(jnp.int32, 16)
    ones = jnp.full((16,), 1, jnp.int32)
    zeros = jnp.zeros((16,), jnp.int32)
    sems = (sem0, sem1, sem2, sem3)
    row0 = wid * ROWS_PER_W

    pltpu.sync_copy(x_hbm.at[:, pl.ds(row0, ROWS_PER_W)], xbuf)

    for b in range(NBUF):
        def zero_body(k, carry, b=b):
            buf[b, k >> 3, pl.ds((k & 7) * 16, 16)] = zeros
            return carry

        lax.fori_loop(0, BC * BR // 16, zero_body, 0)

    def scatter(c, b, val):
        sg = c >> 2
        rt = c & 3
        bsplat = jnp.full((16,), b, jnp.int32)
        for ii in range(2):
            fvec = jnp.full((16,), 2 * sg + ii, jnp.int32)
            for g in range(8):
                rloc = rt * BR + g * 16 + lanes
                xv = plsc.load_gather(xbuf, [fvec, rloc])
                plsc.store_scatter(
                    buf, [bsplat, ii * FIELD_N + xv, g * 16 + lanes], val)

    def start_stream(c, b):
        sg = c >> 2
        rt = c & 3
        pltpu.async_copy(
            buf.at[b],
            out_hbm.at[pl.ds(sg * BC, BC), pl.ds(row0 + rt * BR, BR)],
            sems[b])

    def wait_stream(b):
        pltpu.make_async_copy(
            buf.at[b],
            out_hbm.at[pl.ds(0, BC), pl.ds(0, BR)],
            sems[b]).wait()

    for b in range(NBUF):
        scatter(jnp.int32(b), b, ones)
        start_stream(jnp.int32(b), b)

    def loop_body(it, carry):
        for b in range(NBUF):
            c = it * NBUF + b
            wait_stream(b)
            scatter(c - NBUF, b, zeros)
            scatter(c, b, ones)
            start_stream(c, b)
        return carry

    lax.fori_loop(1, CHUNKS // NBUF, loop_body, 0)

    for b in range(NBUF):
        wait_stream(b)


@jax.jit
def _run(xt):
    mesh = plsc.VectorSubcoreMesh(core_axis_name="c", subcore_axis_name="s")
    f = pl.kernel(
        _body,
        out_type=jax.ShapeDtypeStruct((ROW_W, N_ROWS), jnp.int32),
        mesh=mesh,
        scratch_types=[
            pltpu.VMEM((N_FIELDS, ROWS_PER_W), jnp.int32),
            pltpu.VMEM((NBUF, BC, BR), jnp.int32),
            pltpu.SemaphoreType.DMA,
            pltpu.SemaphoreType.DMA,
            pltpu.SemaphoreType.DMA,
            pltpu.SemaphoreType.DMA,
        ],
        compiler_params=pltpu.CompilerParams(needs_layout_passes=False),
    )
    return f(xt)


def kernel(x):
    return jnp.transpose(_run(jnp.transpose(x)))


# speedup vs baseline: 1.1919x; 1.0129x over previous
import jax
import jax.numpy as jnp
from jax import lax
from jax.experimental import pallas as pl
from jax.experimental.pallas import tpu as pltpu
from jax.experimental.pallas import tpu_sc as plsc

N_ROWS = 16384
N_FIELDS = 26
FIELD_N = 100
ROW_W = N_FIELDS * FIELD_N
NC, NS = 2, 16
NW = NC * NS
ROWS_PER_W = N_ROWS // NW
BC = 200
BR = 128
SGS = ROW_W // BC
RTS = ROWS_PER_W // BR
CHUNKS = SGS * RTS
XW = ROWS_PER_W * N_FIELDS


NBUF = 2


def _body(x_hbm, out_hbm, xbuf, buf, sem0, sem1, sem2, sem3):
    wid = lax.axis_index("c") * NS + lax.axis_index("s")
    lanes = lax.iota(jnp.int32, 16)
    ones = jnp.full((16,), 1, jnp.int32)
    zeros = jnp.zeros((16,), jnp.int32)
    sems = (sem0, sem1, sem2, sem3)
    row0 = wid * ROWS_PER_W

    xcopy = pltpu.async_copy(x_hbm.at[:, pl.ds(row0, ROWS_PER_W)], xbuf, sem0)
    for b in range(NBUF):
        def zero_body(k, carry, b=b):
            buf[b, k >> 3, pl.ds((k & 7) * 16, 16)] = zeros
            return carry

        lax.fori_loop(0, BC * BR // 16, zero_body, 0)
    xcopy.wait()

    def scatter(c, b, val):
        sg = c >> 2
        rt = c & 3
        bsplat = jnp.full((16,), b, jnp.int32)
        for ii in range(2):
            fvec = jnp.full((16,), 2 * sg + ii, jnp.int32)
            for g in range(8):
                rloc = rt * BR + g * 16 + lanes
                xv = plsc.load_gather(xbuf, [fvec, rloc])
                plsc.store_scatter(
                    buf, [bsplat, ii * FIELD_N + xv, g * 16 + lanes], val)

    def start_stream(c, b):
        sg = c >> 2
        rt = c & 3
        pltpu.async_copy(
            buf.at[b],
            out_hbm.at[pl.ds(sg * BC, BC), pl.ds(row0 + rt * BR, BR)],
            sems[b])

    def wait_stream(b):
        pltpu.make_async_copy(
            buf.at[b],
            out_hbm.at[pl.ds(0, BC), pl.ds(0, BR)],
            sems[b]).wait()

    for b in range(NBUF):
        scatter(jnp.int32(b), b, ones)
        start_stream(jnp.int32(b), b)

    def loop_body(it, carry):
        for b in range(NBUF):
            c = it * NBUF + b
            wait_stream(b)
            scatter(c - NBUF, b, zeros)
            scatter(c, b, ones)
            start_stream(c, b)
        return carry

    lax.fori_loop(1, CHUNKS // NBUF, loop_body, 0)

    for b in range(NBUF):
        wait_stream(b)


@jax.jit
def _run(xt):
    mesh = plsc.VectorSubcoreMesh(core_axis_name="c", subcore_axis_name="s")
    f = pl.kernel(
        _body,
        out_type=jax.ShapeDtypeStruct((ROW_W, N_ROWS), jnp.int32),
        mesh=mesh,
        scratch_types=[
            pltpu.VMEM((N_FIELDS, ROWS_PER_W), jnp.int32),
            pltpu.VMEM((NBUF, BC, BR), jnp.int32),
            pltpu.SemaphoreType.DMA,
            pltpu.SemaphoreType.DMA,
            pltpu.SemaphoreType.DMA,
            pltpu.SemaphoreType.DMA,
        ],
        compiler_params=pltpu.CompilerParams(needs_layout_passes=False),
    )
    return f(xt)


def kernel(x):
    return jnp.transpose(_run(jnp.transpose(x)))


# speedup vs baseline: 1.2162x; 1.0204x over previous
import jax
import jax.numpy as jnp
from jax import lax
from jax.experimental import pallas as pl
from jax.experimental.pallas import tpu as pltpu
from jax.experimental.pallas import tpu_sc as plsc

N_ROWS = 16384
N_FIELDS = 26
FIELD_N = 100
ROW_W = N_FIELDS * FIELD_N
NC, NS = 2, 16
NW = NC * NS
ROWS_PER_W = N_ROWS // NW
BC = 200
BR = 128
SGS = ROW_W // BC
RTS = ROWS_PER_W // BR
CHUNKS = SGS * RTS
XW = ROWS_PER_W * N_FIELDS


NBUF = 2


def _body(x_hbm, out_hbm, xbuf, buf, sem0, sem1, sem2, sem3):
    wid = lax.axis_index("c") * NS + lax.axis_index("s")
    lanes = lax.iota(jnp.int32, 16)
    ones = jnp.full((16,), 1, jnp.int32)
    zeros = jnp.zeros((16,), jnp.int32)
    sems = (sem0, sem1, sem2, sem3)

    xcopies = [
        pltpu.async_copy(
            x_hbm.at[:, pl.ds((rt * NW + wid) * BR, BR)],
            xbuf.at[:, pl.ds(rt * BR, BR)],
            sem0)
        for rt in range(RTS)
    ]
    for b in range(NBUF):
        def zero_body(k, carry, b=b):
            buf[b, k >> 3, pl.ds((k & 7) * 16, 16)] = zeros
            return carry

        lax.fori_loop(0, BC * BR // 16, zero_body, 0)
    for c in xcopies:
        c.wait()

    def scatter(c, b, val):
        sg = c >> 2
        rt = c & 3
        bsplat = jnp.full((16,), b, jnp.int32)
        for ii in range(2):
            fvec = jnp.full((16,), 2 * sg + ii, jnp.int32)
            for g in range(8):
                rloc = rt * BR + g * 16 + lanes
                xv = plsc.load_gather(xbuf, [fvec, rloc])
                plsc.store_scatter(
                    buf, [bsplat, ii * FIELD_N + xv, g * 16 + lanes], val)

    def start_stream(c, b):
        sg = c >> 2
        rt = c & 3
        pltpu.async_copy(
            buf.at[b],
            out_hbm.at[pl.ds(sg * BC, BC), pl.ds((rt * NW + wid) * BR, BR)],
            sems[b])

    def wait_stream(b):
        pltpu.make_async_copy(
            buf.at[b],
            out_hbm.at[pl.ds(0, BC), pl.ds(0, BR)],
            sems[b]).wait()

    for b in range(NBUF):
        scatter(jnp.int32(b), b, ones)
        start_stream(jnp.int32(b), b)

    def loop_body(it, carry):
        for b in range(NBUF):
            c = it * NBUF + b
            wait_stream(b)
            scatter(c - NBUF, b, zeros)
            scatter(c, b, ones)
            start_stream(c, b)
        return carry

    lax.fori_loop(1, CHUNKS // NBUF, loop_body, 0)

    for b in range(NBUF):
        wait_stream(b)


@jax.jit
def _run(xt):
    mesh = plsc.VectorSubcoreMesh(core_axis_name="c", subcore_axis_name="s")
    f = pl.kernel(
        _body,
        out_type=jax.ShapeDtypeStruct((ROW_W, N_ROWS), jnp.int32),
        mesh=mesh,
        scratch_types=[
            pltpu.VMEM((N_FIELDS, ROWS_PER_W), jnp.int32),
            pltpu.VMEM((NBUF, BC, BR), jnp.int32),
            pltpu.SemaphoreType.DMA,
            pltpu.SemaphoreType.DMA,
            pltpu.SemaphoreType.DMA,
            pltpu.SemaphoreType.DMA,
        ],
        compiler_params=pltpu.CompilerParams(needs_layout_passes=False),
    )
    return f(xt)


def kernel(x):
    return jnp.transpose(_run(jnp.transpose(x)))


# speedup vs baseline: 1.2167x; 1.0004x over previous
import jax
import jax.numpy as jnp
from jax import lax
from jax.experimental import pallas as pl
from jax.experimental.pallas import tpu as pltpu
from jax.experimental.pallas import tpu_sc as plsc

N_ROWS = 16384
N_FIELDS = 26
FIELD_N = 100
ROW_W = N_FIELDS * FIELD_N
NC, NS = 2, 16
NW = NC * NS
ROWS_PER_W = N_ROWS // NW
BC = 200
BR = 128
SGS = ROW_W // BC
RTS = ROWS_PER_W // BR
CHUNKS = SGS * RTS
XW = ROWS_PER_W * N_FIELDS


NBUF = 2


def _body(x_hbm, out_hbm, xbuf, buf, sem0, sem1, sem2, sem3):
    wid = lax.axis_index("c") * NS + lax.axis_index("s")
    lanes = lax.iota(jnp.int32, 16)
    ones = jnp.full((16,), 1, jnp.int32)
    zeros = jnp.zeros((16,), jnp.int32)
    sems = (sem0, sem1, sem2, sem3)

    xcopies = [
        pltpu.async_copy(
            x_hbm.at[:, pl.ds((rt * NW + wid) * BR, BR)],
            xbuf.at[:, pl.ds(rt * BR, BR)],
            sem0)
        for rt in range(RTS)
    ]
    for b in range(NBUF):
        def zero_body(k, carry, b=b):
            buf[b, k >> 3, pl.ds((k & 7) * 16, 16)] = zeros
            return carry

        lax.fori_loop(0, BC * BR // 16, zero_body, 0)
    for c in xcopies:
        c.wait()

    def scatter(c, b, val):
        sg = c >> 2
        rt = c & 3
        bsplat = jnp.full((16,), b, jnp.int32)
        for ii in range(2):
            fvec = jnp.full((16,), 2 * sg + ii, jnp.int32)
            for g in range(8):
                rloc = rt * BR + g * 16 + lanes
                xv = plsc.load_gather(xbuf, [fvec, rloc])
                plsc.store_scatter(
                    buf, [bsplat, ii * FIELD_N + xv, g * 16 + lanes], val)

    def start_stream(c, b):
        sg = c >> 2
        rt = c & 3
        pltpu.async_copy(
            buf.at[b],
            out_hbm.at[pl.ds(sg * BC, BC), pl.ds((rt * NW + wid) * BR, BR)],
            sems[b])

    def wait_stream(b):
        pltpu.make_async_copy(
            buf.at[b],
            out_hbm.at[pl.ds(0, BC), pl.ds(0, BR)],
            sems[b]).wait()

    for b in range(NBUF):
        scatter(jnp.int32(b), b, ones)
        start_stream(jnp.int32(b), b)

    def loop_body(it, carry):
        for b in range(NBUF):
            c = it * NBUF + b
            wait_stream(b)
            scatter(c - NBUF, b, zeros)
            scatter(c, b, ones)
            start_stream(c, b)
        return carry

    lax.fori_loop(1, CHUNKS // NBUF, loop_body, 0)

    for b in range(NBUF):
        wait_stream(b)


@jax.jit
def _run(xt):
    mesh = plsc.VectorSubcoreMesh(core_axis_name="c", subcore_axis_name="s")
    f = pl.kernel(
        _body,
        out_type=jax.ShapeDtypeStruct((ROW_W, N_ROWS), jnp.int32),
        mesh=mesh,
        scratch_types=[
            pltpu.VMEM((N_FIELDS, ROWS_PER_W), jnp.int32),
            pltpu.VMEM((NBUF, BC, BR), jnp.int32),
            pltpu.SemaphoreType.DMA,
            pltpu.SemaphoreType.DMA,
            pltpu.SemaphoreType.DMA,
            pltpu.SemaphoreType.DMA,
        ],
        compiler_params=pltpu.CompilerParams(
            needs_layout_passes=False, skip_device_barrier=True),
    )
    return f(xt)


def kernel(x):
    return jnp.transpose(_run(jnp.transpose(x)))


# speedup vs baseline: 1.2195x; 1.0023x over previous
import jax
import jax.numpy as jnp
from jax import lax
from jax.experimental import pallas as pl
from jax.experimental.pallas import tpu as pltpu
from jax.experimental.pallas import tpu_sc as plsc

N_ROWS = 16384
N_FIELDS = 26
FIELD_N = 100
ROW_W = N_FIELDS * FIELD_N
NC, NS = 2, 16
NW = NC * NS
ROWS_PER_W = N_ROWS // NW
BC = 200
BR = 128
SGS = ROW_W // BC
RTS = ROWS_PER_W // BR
CHUNKS = SGS * RTS
XW = ROWS_PER_W * N_FIELDS


NBUF = 2


def _body(x_hbm, out_hbm, xbuf, buf, sem0, sem1):
    wid = lax.axis_index("c") * NS + lax.axis_index("s")
    lanes = lax.iota(jnp.int32, 16)
    ones = jnp.full((16,), 1, jnp.int32)
    zeros = jnp.zeros((16,), jnp.int32)
    sems = (sem0, sem1)

    xcopies = [
        pltpu.async_copy(
            x_hbm.at[:, pl.ds((rt * NW + wid) * BR, BR)],
            xbuf.at[:, pl.ds(rt * BR, BR)],
            sem0)
        for rt in range(RTS)
    ]
    for b in range(NBUF):
        def zero_body(k, carry, b=b):
            buf[b, k >> 3, pl.ds((k & 7) * 16, 16)] = zeros
            return carry

        lax.fori_loop(0, BC * BR // 16, zero_body, 0)
    for c in xcopies:
        c.wait()

    def scatter(c, b, val):
        sg = c >> 2
        rt = c & 3
        bsplat = jnp.full((16,), b, jnp.int32)
        for ii in range(2):
            fvec = jnp.full((16,), 2 * sg + ii, jnp.int32)
            for g in range(8):
                rloc = rt * BR + g * 16 + lanes
                xv = plsc.load_gather(xbuf, [fvec, rloc])
                plsc.store_scatter(
                    buf, [bsplat, ii * FIELD_N + xv, g * 16 + lanes], val)

    def start_stream(c, b):
        sg = c >> 2
        rt = c & 3
        pltpu.async_copy(
            buf.at[b],
            out_hbm.at[pl.ds(sg * BC, BC), pl.ds((rt * NW + wid) * BR, BR)],
            sems[b])

    def wait_stream(b):
        pltpu.make_async_copy(
            buf.at[b],
            out_hbm.at[pl.ds(0, BC), pl.ds(0, BR)],
            sems[b]).wait()

    for b in range(NBUF):
        scatter(jnp.int32(b), b, ones)
        start_stream(jnp.int32(b), b)

    def loop_body(it, carry):
        for b in range(NBUF):
            c = it * NBUF + b
            wait_stream(b)
            scatter(c - NBUF, b, zeros)
            scatter(c, b, ones)
            start_stream(c, b)
        return carry

    lax.fori_loop(1, CHUNKS // NBUF, loop_body, 0)

    for b in range(NBUF):
        wait_stream(b)


@jax.jit
def _run(xt):
    mesh = plsc.VectorSubcoreMesh(core_axis_name="c", subcore_axis_name="s")
    f = pl.kernel(
        _body,
        out_type=jax.ShapeDtypeStruct((ROW_W, N_ROWS), jnp.int32),
        mesh=mesh,
        scratch_types=[
            pltpu.VMEM((N_FIELDS, ROWS_PER_W), jnp.int32),
            pltpu.VMEM((NBUF, BC, BR), jnp.int32),
            pltpu.SemaphoreType.DMA,
            pltpu.SemaphoreType.DMA,
        ],
        compiler_params=pltpu.CompilerParams(needs_layout_passes=False),
    )
    return f(xt)


def kernel(x):
    return jnp.transpose(_run(jnp.transpose(x)))
